# baseline (device time: 190963 ns/iter reference)
import jax
import jax.numpy as jnp
from jax import lax
from jax.experimental import pallas as pl
from jax.experimental.pallas import tpu as pltpu

N_DEV = 4


def _ring_allreduce_body(x_ref, out_ref, comm_ref, send_sems, recv_sems):
    my_pos = lax.axis_index("i")
    left = lax.rem(my_pos - 1 + N_DEV, N_DEV)
    right = lax.rem(my_pos + 1, N_DEV)

    barrier_sem = pltpu.get_barrier_semaphore()
    for nbr in (left, right):
        pl.semaphore_signal(
            barrier_sem, inc=1,
            device_id=(nbr,), device_id_type=pl.DeviceIdType.MESH,
        )
    pl.semaphore_wait(barrier_sem, 2)

    out_ref[...] = x_ref[...]
    comm_ref[0, :, :] = x_ref[...]

    for h in range(N_DEV - 1):
        send_slot = h % 2
        recv_slot = (h + 1) % 2
        rdma = pltpu.make_async_remote_copy(
            src_ref=comm_ref.at[send_slot],
            dst_ref=comm_ref.at[recv_slot],
            send_sem=send_sems.at[send_slot],
            recv_sem=recv_sems.at[recv_slot],
            device_id=(right,),
            device_id_type=pl.DeviceIdType.MESH,
        )
        rdma.start()
        rdma.wait()
        out_ref[...] += comm_ref[recv_slot, :, :]


def kernel(ids, E):
    v_loc, d = E.shape
    t = ids.shape[0]

    my_pos = lax.axis_index("i")
    local_ids = ids - my_pos * v_loc
    in_range = (local_ids >= 0) & (local_ids < v_loc)
    safe_ids = jnp.where(in_range, local_ids, 0)
    partial = jnp.where(
        in_range[:, None], jnp.take(E, safe_ids, axis=0), jnp.float32(0)
    )

    return pl.pallas_call(
        _ring_allreduce_body,
        out_shape=jax.ShapeDtypeStruct((t, d), jnp.float32),
        in_specs=[pl.BlockSpec(memory_space=pltpu.VMEM)],
        out_specs=pl.BlockSpec(memory_space=pltpu.VMEM),
        scratch_shapes=[
            pltpu.VMEM((2, t, d), jnp.float32),
            pltpu.SemaphoreType.DMA((2,)),
            pltpu.SemaphoreType.DMA((2,)),
        ],
        compiler_params=pltpu.CompilerParams(collective_id=0),
    )(partial)


# device time: 90143 ns/iter; 2.1184x vs baseline; 2.1184x over previous
import jax
import jax.numpy as jnp
from jax import lax
from jax.experimental import pallas as pl
from jax.experimental.pallas import tpu as pltpu

N_DEV = 4


def _butterfly_allreduce_body(x_ref, out_ref, scratch, send_sems, recv_sems):
    i = lax.axis_index("i")
    pa = i ^ 1
    pb = 3 - i

    barrier_sem = pltpu.get_barrier_semaphore()
    for nbr in (pa, pb):
        pl.semaphore_signal(
            barrier_sem, inc=1,
            device_id=(nbr,), device_id_type=pl.DeviceIdType.MESH,
        )
    pl.semaphore_wait(barrier_sem, 2)

    h_a = ((i == 1) | (i == 2)).astype(jnp.int32)
    w_a = i // 2
    h_b = i // 2
    w_b = i % 2

    a_half = h_a * 256
    a_oth = (1 - h_a) * 256
    a_q = a_half + w_a * 128
    a_sq = a_half + (1 - w_a) * 128
    b_half = 512 + h_b * 256
    b_oth = 512 + (1 - h_b) * 256
    b_q = b_half + w_b * 128
    b_sq = b_half + (1 - w_b) * 128

    out_ref[pl.ds(a_half, 256), :] = x_ref[pl.ds(a_half, 256), :]
    out_ref[pl.ds(b_half, 256), :] = x_ref[pl.ds(b_half, 256), :]

    def exch(k, partner, src, dst):
        return pltpu.make_async_remote_copy(
            src_ref=src, dst_ref=dst,
            send_sem=send_sems.at[k], recv_sem=recv_sems.at[k],
            device_id=(partner,), device_id_type=pl.DeviceIdType.MESH,
        )

    r0 = exch(0, pa, x_ref.at[pl.ds(a_oth, 256)], scratch.at[pl.ds(0, 256)])
    r1 = exch(1, pb, x_ref.at[pl.ds(b_oth, 256)], scratch.at[pl.ds(256, 256)])
    r0.start()
    r1.start()
    r0.wait()
    r1.wait()
    out_ref[pl.ds(a_half, 256), :] += scratch[pl.ds(0, 256), :]
    out_ref[pl.ds(b_half, 256), :] += scratch[pl.ds(256, 256), :]

    r2 = exch(2, pb, out_ref.at[pl.ds(a_sq, 128)], scratch.at[pl.ds(512, 128)])
    r3 = exch(3, pa, out_ref.at[pl.ds(b_sq, 128)], scratch.at[pl.ds(640, 128)])
    r2.start()
    r3.start()
    r2.wait()
    r3.wait()
    out_ref[pl.ds(a_q, 128), :] += scratch[pl.ds(512, 128), :]
    out_ref[pl.ds(b_q, 128), :] += scratch[pl.ds(640, 128), :]

    r4 = exch(4, pb, out_ref.at[pl.ds(a_q, 128)], out_ref.at[pl.ds(a_q, 128)])
    r5 = exch(5, pa, out_ref.at[pl.ds(b_q, 128)], out_ref.at[pl.ds(b_q, 128)])
    r4.start()
    r5.start()
    r4.wait()
    r5.wait()

    r6 = exch(6, pa, out_ref.at[pl.ds(a_half, 256)], out_ref.at[pl.ds(a_half, 256)])
    r7 = exch(7, pb, out_ref.at[pl.ds(b_half, 256)], out_ref.at[pl.ds(b_half, 256)])
    r6.start()
    r7.start()
    r6.wait()
    r7.wait()


def kernel(ids, E):
    v_loc, d = E.shape
    t = ids.shape[0]

    my_pos = lax.axis_index("i")
    local_ids = ids - my_pos * v_loc
    in_range = (local_ids >= 0) & (local_ids < v_loc)
    safe_ids = jnp.where(in_range, local_ids, 0)
    partial = jnp.where(
        in_range[:, None], jnp.take(E, safe_ids, axis=0), jnp.float32(0)
    )

    return pl.pallas_call(
        _butterfly_allreduce_body,
        out_shape=jax.ShapeDtypeStruct((t, d), jnp.float32),
        in_specs=[pl.BlockSpec(memory_space=pltpu.VMEM)],
        out_specs=pl.BlockSpec(memory_space=pltpu.VMEM),
        scratch_shapes=[
            pltpu.VMEM((768, d), jnp.float32),
            pltpu.SemaphoreType.DMA((8,)),
            pltpu.SemaphoreType.DMA((8,)),
        ],
        compiler_params=pltpu.CompilerParams(collective_id=0),
    )(partial)


# device time: 78847 ns/iter; 2.4219x vs baseline; 1.1433x over previous
import jax
import jax.numpy as jnp
from jax import lax
from jax.experimental import pallas as pl
from jax.experimental.pallas import tpu as pltpu

N_DEV = 4
T = 1024
UNROLL = 8


def _body(ids_ref, e_ref, mask_ref, out_ref,
          partial, scratch, send_sems, recv_sems, gather_sem):
    i = lax.axis_index("i")
    pa = i ^ 1
    pb = 3 - i

    def issue_chunk(c, carry):
        base = c * UNROLL
        for u in range(UNROLL):
            t = base + u
            idx = ids_ref[t]
            pltpu.make_async_copy(e_ref.at[idx], partial.at[t], gather_sem).start()
        return carry

    lax.fori_loop(0, T // UNROLL, issue_chunk, 0)

    barrier_sem = pltpu.get_barrier_semaphore()
    for nbr in (pa, pb):
        pl.semaphore_signal(
            barrier_sem, inc=1,
            device_id=(nbr,), device_id_type=pl.DeviceIdType.MESH,
        )
    pl.semaphore_wait(barrier_sem, 2)

    def wait_chunk(c, carry):
        for _ in range(UNROLL):
            pltpu.make_async_copy(e_ref.at[0], partial.at[0], gather_sem).wait()
        return carry

    lax.fori_loop(0, T // UNROLL, wait_chunk, 0)

    partial[...] = partial[...] * mask_ref[...]

    h_a = ((i == 1) | (i == 2)).astype(jnp.int32)
    w_a = i // 2
    h_b = i // 2
    w_b = i % 2

    a_half = h_a * 256
    a_oth = (1 - h_a) * 256
    a_q = a_half + w_a * 128
    a_sq = a_half + (1 - w_a) * 128
    b_half = 512 + h_b * 256
    b_oth = 512 + (1 - h_b) * 256
    b_q = b_half + w_b * 128
    b_sq = b_half + (1 - w_b) * 128

    out_ref[pl.ds(a_half, 256), :] = partial[pl.ds(a_half, 256), :]
    out_ref[pl.ds(b_half, 256), :] = partial[pl.ds(b_half, 256), :]

    def exch(k, partner, src, dst):
        return pltpu.make_async_remote_copy(
            src_ref=src, dst_ref=dst,
            send_sem=send_sems.at[k], recv_sem=recv_sems.at[k],
            device_id=(partner,), device_id_type=pl.DeviceIdType.MESH,
        )

    r0 = exch(0, pa, partial.at[pl.ds(a_oth, 256)], scratch.at[pl.ds(0, 256)])
    r1 = exch(1, pb, partial.at[pl.ds(b_oth, 256)], scratch.at[pl.ds(256, 256)])
    r0.start()
    r1.start()
    r0.wait()
    r1.wait()
    out_ref[pl.ds(a_half, 256), :] += scratch[pl.ds(0, 256), :]
    out_ref[pl.ds(b_half, 256), :] += scratch[pl.ds(256, 256), :]

    r2 = exch(2, pb, out_ref.at[pl.ds(a_sq, 128)], scratch.at[pl.ds(512, 128)])
    r3 = exch(3, pa, out_ref.at[pl.ds(b_sq, 128)], scratch.at[pl.ds(640, 128)])
    r2.start()
    r3.start()
    r2.wait()
    r3.wait()
    out_ref[pl.ds(a_q, 128), :] += scratch[pl.ds(512, 128), :]
    out_ref[pl.ds(b_q, 128), :] += scratch[pl.ds(640, 128), :]

    r4 = exch(4, pb, out_ref.at[pl.ds(a_q, 128)], out_ref.at[pl.ds(a_q, 128)])
    r5 = exch(5, pa, out_ref.at[pl.ds(b_q, 128)], out_ref.at[pl.ds(b_q, 128)])
    r4.start()
    r5.start()
    r4.wait()
    r5.wait()

    r6 = exch(6, pa, out_ref.at[pl.ds(a_half, 256)], out_ref.at[pl.ds(a_half, 256)])
    r7 = exch(7, pb, out_ref.at[pl.ds(b_half, 256)], out_ref.at[pl.ds(b_half, 256)])
    r6.start()
    r7.start()
    r6.wait()
    r7.wait()


def kernel(ids, E):
    v_loc, d = E.shape

    my_pos = lax.axis_index("i")
    local_ids = ids - my_pos * v_loc
    in_range = (local_ids >= 0) & (local_ids < v_loc)
    safe_ids = jnp.where(in_range, local_ids, 0)
    mask = in_range.astype(jnp.float32)[:, None]

    grid_spec = pltpu.PrefetchScalarGridSpec(
        num_scalar_prefetch=1,
        grid=(),
        in_specs=[
            pl.BlockSpec(memory_space=pltpu.MemorySpace.HBM),
            pl.BlockSpec(memory_space=pltpu.VMEM),
        ],
        out_specs=pl.BlockSpec(memory_space=pltpu.VMEM),
        scratch_shapes=[
            pltpu.VMEM((T, d), jnp.float32),
            pltpu.VMEM((768, d), jnp.float32),
            pltpu.SemaphoreType.DMA((8,)),
            pltpu.SemaphoreType.DMA((8,)),
            pltpu.SemaphoreType.DMA,
        ],
    )
    return pl.pallas_call(
        _body,
        grid_spec=grid_spec,
        out_shape=jax.ShapeDtypeStruct((T, d), jnp.float32),
        compiler_params=pltpu.CompilerParams(collective_id=0),
    )(safe_ids, E, mask)


# device time: 60412 ns/iter; 3.1610x vs baseline; 1.3052x over previous
import jax
import jax.numpy as jnp
from jax import lax
from jax.experimental import pallas as pl
from jax.experimental.pallas import tpu as pltpu

N_DEV = 4
T = 1024
UNROLL = 8


def _body(packed_ref, n_ref, e_ref, out_ref,
          partial, scratch, send_sems, recv_sems, gather_sem):
    i = lax.axis_index("i")
    pa = i ^ 1
    pb = 3 - i
    n_mine = n_ref[0]

    partial[...] = jnp.zeros((T, partial.shape[1]), jnp.float32)

    def issue(k, carry):
        v = packed_ref[k]
        tok = v >> 14
        row = v & 16383
        pltpu.make_async_copy(e_ref.at[row], partial.at[tok], gather_sem).start()
        return carry

    lax.fori_loop(0, n_mine, issue, 0)

    barrier_sem = pltpu.get_barrier_semaphore()
    for nbr in (pa, pb):
        pl.semaphore_signal(
            barrier_sem, inc=1,
            device_id=(nbr,), device_id_type=pl.DeviceIdType.MESH,
        )
    pl.semaphore_wait(barrier_sem, 2)

    def drain(k, carry):
        pltpu.make_async_copy(e_ref.at[0], partial.at[0], gather_sem).wait()
        return carry

    lax.fori_loop(0, n_mine, drain, 0)

    h_a = ((i == 1) | (i == 2)).astype(jnp.int32)
    w_a = i // 2
    h_b = i // 2
    w_b = i % 2

    a_half = h_a * 256
    a_oth = (1 - h_a) * 256
    a_q = a_half + w_a * 128
    a_sq = a_half + (1 - w_a) * 128
    b_half = 512 + h_b * 256
    b_oth = 512 + (1 - h_b) * 256
    b_q = b_half + w_b * 128
    b_sq = b_half + (1 - w_b) * 128

    out_ref[pl.ds(a_half, 256), :] = partial[pl.ds(a_half, 256), :]
    out_ref[pl.ds(b_half, 256), :] = partial[pl.ds(b_half, 256), :]

    def exch(k, partner, src, dst):
        return pltpu.make_async_remote_copy(
            src_ref=src, dst_ref=dst,
            send_sem=send_sems.at[k], recv_sem=recv_sems.at[k],
            device_id=(partner,), device_id_type=pl.DeviceIdType.MESH,
        )

    r0 = exch(0, pa, partial.at[pl.ds(a_oth, 256)], scratch.at[pl.ds(0, 256)])
    r1 = exch(1, pb, partial.at[pl.ds(b_oth, 256)], scratch.at[pl.ds(256, 256)])
    r0.start()
    r1.start()
    r0.wait()
    r1.wait()
    out_ref[pl.ds(a_half, 256), :] += scratch[pl.ds(0, 256), :]
    out_ref[pl.ds(b_half, 256), :] += scratch[pl.ds(256, 256), :]

    r2 = exch(2, pb, out_ref.at[pl.ds(a_sq, 128)], scratch.at[pl.ds(512, 128)])
    r3 = exch(3, pa, out_ref.at[pl.ds(b_sq, 128)], scratch.at[pl.ds(640, 128)])
    r2.start()
    r3.start()
    r2.wait()
    r3.wait()
    out_ref[pl.ds(a_q, 128), :] += scratch[pl.ds(512, 128), :]
    out_ref[pl.ds(b_q, 128), :] += scratch[pl.ds(640, 128), :]

    r4 = exch(4, pb, out_ref.at[pl.ds(a_q, 128)], out_ref.at[pl.ds(a_q, 128)])
    r5 = exch(5, pa, out_ref.at[pl.ds(b_q, 128)], out_ref.at[pl.ds(b_q, 128)])
    r4.start()
    r5.start()
    r4.wait()
    r5.wait()

    r6 = exch(6, pa, out_ref.at[pl.ds(a_half, 256)], out_ref.at[pl.ds(a_half, 256)])
    r7 = exch(7, pb, out_ref.at[pl.ds(b_half, 256)], out_ref.at[pl.ds(b_half, 256)])
    r6.start()
    r7.start()
    r6.wait()
    r7.wait()


def kernel(ids, E):
    v_loc, d = E.shape

    my_pos = lax.axis_index("i")
    local_ids = ids - my_pos * v_loc
    in_range = (local_ids >= 0) & (local_ids < v_loc)
    tok = jnp.arange(T, dtype=jnp.int32)
    key = jnp.where(in_range, tok, T + tok)
    packed = jnp.sort((key << 14) | jnp.where(in_range, local_ids, 0))
    n_mine = jnp.sum(in_range.astype(jnp.int32)).reshape(1)

    grid_spec = pltpu.PrefetchScalarGridSpec(
        num_scalar_prefetch=2,
        grid=(),
        in_specs=[
            pl.BlockSpec(memory_space=pltpu.MemorySpace.HBM),
        ],
        out_specs=pl.BlockSpec(memory_space=pltpu.VMEM),
        scratch_shapes=[
            pltpu.VMEM((T, d), jnp.float32),
            pltpu.VMEM((768, d), jnp.float32),
            pltpu.SemaphoreType.DMA((8,)),
            pltpu.SemaphoreType.DMA((8,)),
            pltpu.SemaphoreType.DMA,
        ],
    )
    return pl.pallas_call(
        _body,
        grid_spec=grid_spec,
        out_shape=jax.ShapeDtypeStruct((T, d), jnp.float32),
        compiler_params=pltpu.CompilerParams(collective_id=0),
    )(packed, n_mine, E)
